# baseline (speedup 1.0000x reference)
import functools
import math

import jax
import jax.numpy as jnp
from jax.experimental import pallas as pl
from jax.experimental.pallas import tpu as pltpu

NATOM = 128
N_RADIAL = 20
CUTOFF = 5.0


def _mlp_body(q_ref, W1_ref, b1_ref, W2_ref, b2_ref, x_ref):
    h = jnp.dot(q_ref[...], W1_ref[...], preferred_element_type=jnp.float32)
    h = h + b1_ref[...]
    h = h * jax.nn.sigmoid(h)
    x_ref[...] = (
        jnp.dot(h, W2_ref[...], preferred_element_type=jnp.float32) + b2_ref[...]
    )


def _filter_body(ew_ref, Wf_ref, bf_ref, filt_ref, dirs_ref):
    ew = ew_ref[...]
    d2 = jnp.sum(ew * ew, axis=1, keepdims=True)
    d = jnp.sqrt(d2)
    inv_d = 1.0 / d
    n = jax.lax.broadcasted_iota(jnp.int32, (1, N_RADIAL), 1).astype(jnp.float32)
    n = n + 1.0
    basis = jnp.sin(n * (math.pi / CUTOFF) * d) * inv_d
    cutoff_v = 0.5 * (jnp.cos(math.pi / CUTOFF * d) + 1.0)
    cutoff_v = jnp.where(d < CUTOFF, cutoff_v, 0.0)
    filt = jnp.dot(basis, Wf_ref[...], preferred_element_type=jnp.float32)
    filt_ref[...] = (filt + bf_ref[...]) * cutoff_v
    dirs_ref[...] = ew * inv_d


def kernel(q, mu, edge_index, edge_weight, W1, b1, W2, b2, Wf, bf):
    N = q.shape[0]
    E = edge_weight.shape[0]
    BN = 2000
    BE = 4000

    x = pl.pallas_call(
        _mlp_body,
        grid=(N // BN,),
        in_specs=[
            pl.BlockSpec((BN, NATOM), lambda i: (i, 0)),
            pl.BlockSpec((NATOM, NATOM), lambda i: (0, 0)),
            pl.BlockSpec((NATOM,), lambda i: (0,)),
            pl.BlockSpec((NATOM, 3 * NATOM), lambda i: (0, 0)),
            pl.BlockSpec((3 * NATOM,), lambda i: (0,)),
        ],
        out_specs=pl.BlockSpec((BN, 3 * NATOM), lambda i: (i, 0)),
        out_shape=jax.ShapeDtypeStruct((N, 3 * NATOM), jnp.float32),
    )(q, W1, b1, W2, b2)

    filt, dirs = pl.pallas_call(
        _filter_body,
        grid=(E // BE,),
        in_specs=[
            pl.BlockSpec((BE, 3), lambda i: (i, 0)),
            pl.BlockSpec((N_RADIAL, 3 * NATOM), lambda i: (0, 0)),
            pl.BlockSpec((3 * NATOM,), lambda i: (0,)),
        ],
        out_specs=[
            pl.BlockSpec((BE, 3 * NATOM), lambda i: (i, 0)),
            pl.BlockSpec((BE, 3), lambda i: (i, 0)),
        ],
        out_shape=[
            jax.ShapeDtypeStruct((E, 3 * NATOM), jnp.float32),
            jax.ShapeDtypeStruct((E, 3), jnp.float32),
        ],
    )(edge_weight, Wf, bf)

    idx_i = edge_index[0]
    idx_j = edge_index[1]
    xj = x[idx_j]
    muj = mu[idx_j]
    xe = filt * xj
    dq, dmuR, dmumu = jnp.split(xe, 3, axis=-1)
    q_update = jnp.zeros_like(q).at[idx_i].add(dq)
    dmu = dmuR[:, None, :] * dirs[:, :, None] + dmumu[:, None, :] * muj
    mu_update = jnp.zeros_like(mu).at[idx_i].add(dmu)
    return (q + q_update, mu + mu_update)


# speedup vs baseline: 5.2412x; 5.2412x over previous
import functools
import math

import jax
import jax.numpy as jnp
from jax import lax
from jax.experimental import pallas as pl
from jax.experimental.pallas import tpu as pltpu
from jax.experimental.pallas import tpu_sc as plsc

NATOM = 128
N_RADIAL = 20
CUTOFF = 5.0

N_NODES = 10000
N_EDGES = 320000

NC = 2
NS = 16
L = 16

D = 3 * NATOM
FD = 512
NW = NC * NS
NPASS = 3
TR = 128
NPAD = NPASS * NW * TR
ACC_ROWS = TR + L
CHUNK = 4000
NVEC = CHUNK // L
NCHUNK = N_EDGES // CHUNK


def _mlp_body(q_ref, W1_ref, b1_ref, W2_ref, b2_ref, x_ref):
    h = jnp.dot(q_ref[...], W1_ref[...], preferred_element_type=jnp.float32)
    h = h + b1_ref[...]
    h = h * jax.nn.sigmoid(h)
    x_ref[...] = (
        jnp.dot(h, W2_ref[...], preferred_element_type=jnp.float32) + b2_ref[...]
    )


def _filter_body(ew_ref, Wf_ref, bf_ref, filt_ref):
    ew = ew_ref[...]
    d2 = jnp.sum(ew * ew, axis=1, keepdims=True)
    d = jnp.sqrt(d2)
    inv_d = 1.0 / d
    n = jax.lax.broadcasted_iota(jnp.int32, (1, N_RADIAL), 1).astype(jnp.float32)
    n = n + 1.0
    basis = jnp.sin(n * (math.pi / CUTOFF) * d) * inv_d
    cutoff_v = 0.5 * (jnp.cos(math.pi / CUTOFF * d) + 1.0)
    cutoff_v = jnp.where(d < CUTOFF, cutoff_v, 0.0)
    filt = jnp.dot(basis, Wf_ref[...], preferred_element_type=jnp.float32)
    filt_ref[:, 0:D] = (filt + bf_ref[...]) * cutoff_v
    filt_ref[:, D:D + 3] = ew * inv_d
    filt_ref[:, D + 3:FD] = jnp.zeros((ew.shape[0], FD - D - 3), jnp.float32)


def _sc_body(
    filt_hbm, x_hbm, mu_hbm, qpad_hbm, mupad_hbm, ii_hbm, jj_hbm,
    qout_hbm, muout_hbm,
    q_acc, mu_acc, ii_buf, jj_buf, le, li, lj, e16, j16,
    f_buf, x_buf, m_buf,
):
    c = lax.axis_index("c")
    s = lax.axis_index("s")
    w = c * NS + s
    iota = lax.broadcasted_iota(jnp.int32, (L,), 0)

    for p in range(NPASS):
        lo = (p * NW + w) * TR

        grow = pl.multiple_of(lo, TR)
        pltpu.sync_copy(qpad_hbm.at[pl.ds(grow, TR)], q_acc.at[pl.ds(0, TR)])
        pltpu.sync_copy(mupad_hbm.at[pl.ds(grow, TR)], mu_acc.at[pl.ds(0, TR)])

        def chunk_body(ci, _):
            base = pl.multiple_of(ci * CHUNK, CHUNK)
            pltpu.sync_copy(ii_hbm.at[pl.ds(base, CHUNK)], ii_buf)
            pltpu.sync_copy(jj_hbm.at[pl.ds(base, CHUNK)], jj_buf)

            def scan_body(k, cnt):
                o = pl.multiple_of(k * L, L)
                vi = ii_buf[pl.ds(o, L)]
                vj = jj_buf[pl.ds(o, L)]
                ve = base + k * L + iota
                il = vi - lo
                mask = (il >= 0) & (il < TR)
                mi = mask.astype(jnp.int32)
                pos = cnt + plsc.cumsum(mi) - 1
                plsc.store_scatter(le, [pos], ve, mask=mask)
                plsc.store_scatter(li, [pos], il, mask=mask)
                plsc.store_scatter(lj, [pos], vj, mask=mask)
                return cnt + jnp.sum(mi)

            n = lax.fori_loop(0, NVEC, scan_body, jnp.int32(0))

            plsc.store_scatter(le, [n + iota], jnp.zeros((L,), jnp.int32))
            plsc.store_scatter(li, [n + iota], TR + iota)
            plsc.store_scatter(lj, [n + iota], jnp.zeros((L,), jnp.int32))

            nsb = (n + L - 1) // L

            def sb_body(b, _2):
                off = pl.multiple_of(b * L, L)
                sl = pl.ds(off, L)
                e16[...] = le[sl]
                j16[...] = lj[sl]
                ilv = li[sl]
                pltpu.sync_copy(filt_hbm.at[e16], f_buf)
                pltpu.sync_copy(x_hbm.at[j16], x_buf)
                pltpu.sync_copy(mu_hbm.at[j16], m_buf)

                for lane in range(L):
                    rowv = jnp.full((L,), ilv[lane])
                    dv = f_buf[lane, pl.ds(D, L)]
                    d0 = jnp.full((L,), dv[0])
                    d1 = jnp.full((L,), dv[1])
                    d2 = jnp.full((L,), dv[2])

                    @pl.loop(0, NATOM // L)
                    def _feat(k, lane=lane, rowv=rowv, d0=d0, d1=d1, d2=d2):
                        c0 = pl.ds(k * L, L)
                        c1 = pl.ds(NATOM + k * L, L)
                        c2 = pl.ds(2 * NATOM + k * L, L)
                        colv = k * L + iota
                        dq = f_buf[lane, c0] * x_buf[lane, c0]
                        plsc.addupdate_scatter(q_acc, [rowv, colv], dq)
                        a = f_buf[lane, c1] * x_buf[lane, c1]
                        bb = f_buf[lane, c2] * x_buf[lane, c2]
                        plsc.addupdate_scatter(
                            mu_acc, [rowv, colv], a * d0 + bb * m_buf[lane, c0])
                        plsc.addupdate_scatter(
                            mu_acc, [rowv, colv + NATOM], a * d1 + bb * m_buf[lane, c1])
                        plsc.addupdate_scatter(
                            mu_acc, [rowv, colv + 2 * NATOM], a * d2 + bb * m_buf[lane, c2])

                return _2

            lax.fori_loop(0, nsb, sb_body, jnp.int32(0))
            return _

        lax.fori_loop(0, NCHUNK, chunk_body, jnp.int32(0))

        pltpu.sync_copy(q_acc.at[pl.ds(0, TR)], qout_hbm.at[pl.ds(grow, TR)])
        pltpu.sync_copy(mu_acc.at[pl.ds(0, TR)], muout_hbm.at[pl.ds(grow, TR)])


def kernel(q, mu, edge_index, edge_weight, W1, b1, W2, b2, Wf, bf):
    N = q.shape[0]
    E = edge_weight.shape[0]
    BN = 2000
    BE = 4000

    x = pl.pallas_call(
        _mlp_body,
        grid=(N // BN,),
        in_specs=[
            pl.BlockSpec((BN, NATOM), lambda i: (i, 0)),
            pl.BlockSpec((NATOM, NATOM), lambda i: (0, 0)),
            pl.BlockSpec((NATOM,), lambda i: (0,)),
            pl.BlockSpec((NATOM, D), lambda i: (0, 0)),
            pl.BlockSpec((D,), lambda i: (0,)),
        ],
        out_specs=pl.BlockSpec((BN, D), lambda i: (i, 0)),
        out_shape=jax.ShapeDtypeStruct((N, D), jnp.float32),
    )(q, W1, b1, W2, b2)

    filt = pl.pallas_call(
        _filter_body,
        grid=(E // BE,),
        in_specs=[
            pl.BlockSpec((BE, 3), lambda i: (i, 0)),
            pl.BlockSpec((N_RADIAL, D), lambda i: (0, 0)),
            pl.BlockSpec((D,), lambda i: (0,)),
        ],
        out_specs=pl.BlockSpec((BE, FD), lambda i: (i, 0)),
        out_shape=jax.ShapeDtypeStruct((E, FD), jnp.float32),
    )(edge_weight, Wf, bf)

    mu_flat = mu.reshape(N, D)
    qpad = jnp.pad(q, ((0, NPAD - N), (0, 0)))
    mupad = jnp.pad(mu_flat, ((0, NPAD - N), (0, 0)))
    idx_i = edge_index[0]
    idx_j = edge_index[1]

    mesh = plsc.VectorSubcoreMesh(core_axis_name="c", subcore_axis_name="s")
    sc = pl.kernel(
        _sc_body,
        out_type=[
            jax.ShapeDtypeStruct((NPAD, NATOM), jnp.float32),
            jax.ShapeDtypeStruct((NPAD, D), jnp.float32),
        ],
        mesh=mesh,
        compiler_params=pltpu.CompilerParams(needs_layout_passes=False),
        scratch_types=[
            pltpu.VMEM((ACC_ROWS, NATOM), jnp.float32),
            pltpu.VMEM((ACC_ROWS, D), jnp.float32),
            pltpu.VMEM((CHUNK,), jnp.int32),
            pltpu.VMEM((CHUNK,), jnp.int32),
            pltpu.VMEM((CHUNK + L,), jnp.int32),
            pltpu.VMEM((CHUNK + L,), jnp.int32),
            pltpu.VMEM((CHUNK + L,), jnp.int32),
            pltpu.VMEM((L,), jnp.int32),
            pltpu.VMEM((L,), jnp.int32),
            pltpu.VMEM((L, FD), jnp.float32),
            pltpu.VMEM((L, D), jnp.float32),
            pltpu.VMEM((L, D), jnp.float32),
        ],
    )
    q_out, mu_out = sc(filt, x, mu_flat, qpad, mupad, idx_i, idx_j)
    return (q_out[:N], mu_out[:N].reshape(N, 3, NATOM))
